# Initial kernel scaffold; baseline (speedup 1.0000x reference)
import jax
import jax.numpy as jnp
from jax.experimental import pallas as pl


def kernel(x, edge_index, W, b, alpha, ln_gamma, ln_beta):
    raise NotImplementedError("write your pallas kernel here")


# speedup vs baseline: 3.7283x; 3.7283x over previous
import functools

import jax
import jax.numpy as jnp
from jax import lax
from jax.experimental import pallas as pl
from jax.experimental.pallas import tpu as pltpu
from jax.experimental.pallas import tpu_sc as plsc

N_NODES = 10000
N_EDGES = 160000
IN_DIM = 256
OUT_DIM = 256
HALF = 128

CHUNK = 128
TILES = 16
CH_PER_TILE = 79
E_PAD = TILES * CH_PER_TILE * CHUNK
N_PAD = 10240
ROWS_PER_TILE = N_PAD // TILES


def _sc_aggregate(x_lo, x_hi, src_p, dst_p, z_h, z_d):
    mesh = plsc.VectorSubcoreMesh(core_axis_name="c", subcore_axis_name="s")
    f32 = jnp.float32

    @functools.partial(
        pl.kernel,
        out_type=[
            jax.ShapeDtypeStruct((N_PAD, HALF), f32),
            jax.ShapeDtypeStruct((N_PAD, HALF), f32),
            jax.ShapeDtypeStruct((N_PAD,), f32),
            jax.ShapeDtypeStruct((N_PAD,), f32),
        ],
        mesh=mesh,
        scratch_types=[
            pltpu.VMEM((CHUNK,), jnp.int32),
            pltpu.VMEM((CHUNK,), jnp.int32),
            pltpu.VMEM((CHUNK, HALF), f32),
            pltpu.VMEM((CHUNK,), f32),
            pltpu.VMEM_SHARED((N_PAD, HALF), f32),
            pltpu.VMEM_SHARED((N_PAD,), f32),
            pltpu.SemaphoreType.DMA,
        ],
    )
    def agg(xlo_hbm, xhi_hbm, src_hbm, dst_hbm, zh_hbm, zd_hbm,
            h0_out, h1_out, d0_out, d1_out,
            src_v, dst_v, rows_v, ones_v, sh_h, sh_d, sem):
        cid = lax.axis_index("c")
        sid = lax.axis_index("s")
        row0 = sid * ROWS_PER_TILE

        for i in range(CHUNK // 16):
            ones_v[pl.ds(i * 16, 16)] = jnp.ones((16,), f32)

        @pl.when(sid == 0)
        def _():
            pltpu.sync_copy(zh_hbm, sh_h)
            pltpu.sync_copy(zd_hbm, sh_d)

        plsc.subcore_barrier()

        def run(x_hbm, deg_lo):
            def body(i, _):
                base = (sid * CH_PER_TILE + i) * CHUNK
                pltpu.sync_copy(src_hbm.at[pl.ds(base, CHUNK)], src_v)
                pltpu.sync_copy(dst_hbm.at[pl.ds(base, CHUNK)], dst_v)
                pltpu.async_copy(x_hbm.at[src_v], rows_v, sem).wait()
                pltpu.sync_copy(rows_v, sh_h.at[dst_v], add=True)
                dcond = (i < 40) if deg_lo else (i >= 40)

                @pl.when(dcond)
                def _():
                    pltpu.sync_copy(ones_v, sh_d.at[dst_v], add=True)
                return 0
            lax.fori_loop(0, CH_PER_TILE, body, 0)

        @pl.when(cid == 0)
        def _():
            run(xlo_hbm, True)

        @pl.when(cid == 1)
        def _():
            run(xhi_hbm, False)

        plsc.subcore_barrier()

        @pl.when(cid == 0)
        def _():
            pltpu.sync_copy(sh_h.at[pl.ds(row0, ROWS_PER_TILE)],
                            h0_out.at[pl.ds(row0, ROWS_PER_TILE)])
            pltpu.sync_copy(sh_d.at[pl.ds(row0, ROWS_PER_TILE)],
                            d0_out.at[pl.ds(row0, ROWS_PER_TILE)])

        @pl.when(cid == 1)
        def _():
            pltpu.sync_copy(sh_h.at[pl.ds(row0, ROWS_PER_TILE)],
                            h1_out.at[pl.ds(row0, ROWS_PER_TILE)])
            pltpu.sync_copy(sh_d.at[pl.ds(row0, ROWS_PER_TILE)],
                            d1_out.at[pl.ds(row0, ROWS_PER_TILE)])

    return agg(x_lo, x_hi, src_p, dst_p, z_h, z_d)


def _tc_tail(x, h_lo, h_hi, d0, d1, w_lo, w_hi, b2, alpha2, g2, beta2):
    BLK = 1000

    def body(x_ref, h0_ref, h1_ref, d0_ref, d1_ref, wlo_ref, whi_ref,
             b_ref, a_ref, g_ref, bt_ref, o_ref):
        x_blk = x_ref[...]
        deg = d0_ref[...] + d1_ref[...]
        inv = 1.0 / (deg + 1.0)
        pre_lo = (h0_ref[...] + x_blk[:, :HALF]) * inv
        pre_hi = (h1_ref[...] + x_blk[:, HALF:]) * inv
        out = jnp.dot(pre_lo, wlo_ref[...], preferred_element_type=jnp.float32)
        out = out + jnp.dot(pre_hi, whi_ref[...],
                            preferred_element_type=jnp.float32)
        out = out + b_ref[...]
        a = jax.nn.sigmoid(a_ref[0, 0])
        out = out * a + x_blk * (1.0 - a)
        mean = jnp.mean(out, axis=-1, keepdims=True)
        var = jnp.mean((out - mean) ** 2, axis=-1, keepdims=True)
        o_ref[...] = ((out - mean) * lax.rsqrt(var + 1e-5)) * g_ref[...] \
            + bt_ref[...]

    grid = (N_NODES // BLK,)
    return pl.pallas_call(
        body,
        grid=grid,
        in_specs=[
            pl.BlockSpec((BLK, IN_DIM), lambda i: (i, 0)),
            pl.BlockSpec((BLK, HALF), lambda i: (i, 0)),
            pl.BlockSpec((BLK, HALF), lambda i: (i, 0)),
            pl.BlockSpec((BLK, 1), lambda i: (i, 0)),
            pl.BlockSpec((BLK, 1), lambda i: (i, 0)),
            pl.BlockSpec((HALF, OUT_DIM), lambda i: (0, 0)),
            pl.BlockSpec((HALF, OUT_DIM), lambda i: (0, 0)),
            pl.BlockSpec((1, OUT_DIM), lambda i: (0, 0)),
            pl.BlockSpec((1, 1), lambda i: (0, 0)),
            pl.BlockSpec((1, OUT_DIM), lambda i: (0, 0)),
            pl.BlockSpec((1, OUT_DIM), lambda i: (0, 0)),
        ],
        out_specs=pl.BlockSpec((BLK, OUT_DIM), lambda i: (i, 0)),
        out_shape=jax.ShapeDtypeStruct((N_NODES, OUT_DIM), jnp.float32),
    )(x, h_lo, h_hi, d0, d1, w_lo, w_hi, b2, alpha2, g2, beta2)


def kernel(x, edge_index, W, b, alpha, ln_gamma, ln_beta):
    src = edge_index[0].astype(jnp.int32)
    dst = edge_index[1].astype(jnp.int32)
    pad = E_PAD - N_EDGES
    src_p = jnp.concatenate([src, jnp.zeros((pad,), jnp.int32)])
    dst_p = jnp.concatenate([dst, jnp.full((pad,), N_NODES, jnp.int32)])
    x_lo = x[:, :HALF]
    x_hi = x[:, HALF:]
    z_h = jnp.zeros((N_PAD, HALF), jnp.float32)
    z_d = jnp.zeros((N_PAD,), jnp.float32)

    h_lo, h_hi, d0, d1 = _sc_aggregate(x_lo, x_hi, src_p, dst_p, z_h, z_d)
    h_lo = h_lo[:N_NODES]
    h_hi = h_hi[:N_NODES]
    d0 = d0[:N_NODES].reshape(N_NODES, 1)
    d1 = d1[:N_NODES].reshape(N_NODES, 1)

    wt = W.T
    w_lo = wt[:HALF, :]
    w_hi = wt[HALF:, :]
    b2 = b.reshape(1, OUT_DIM)
    alpha2 = alpha.reshape(1, 1)
    g2 = ln_gamma.reshape(1, OUT_DIM)
    beta2 = ln_beta.reshape(1, OUT_DIM)
    return _tc_tail(x, h_lo, h_hi, d0, d1, w_lo, w_hi, b2, alpha2, g2, beta2)
